# baseline (speedup 1.0000x reference)
import jax
import jax.numpy as jnp
import numpy as np
from jax import lax
from jax.experimental import pallas as pl
from jax.experimental.pallas import tpu as pltpu
from jax.experimental.pallas import tpu_sc as plsc

B = 256
B2 = B // 2
NCH = 50000
NN = 10000
FAN = 16

NC = 2
NS = 16
NW = NC * NS
GSZ = 8
GPP = 80
G0 = 40
G1 = GPP - G0
GMAX = max(G0, G1)
N_PAD = 16 * GPP * GSZ
GROW_PAD = 64

R_C = 1280

_BCAST_DNUMS = lax.GatherDimensionNumbers(
    offset_dims=(), collapsed_slice_dims=(0,), start_index_map=(0,))


def _bcast_lane(v, r):
    idx = jnp.full((16, 1), r, jnp.int32)
    return lax.gather(v, idx, _BCAST_DNUMS, (1,),
                      mode=lax.GatherScatterMode.PROMISE_IN_BOUNDS)


def _log_t_body(s_ref, o_ref):
    o_ref[...] = jnp.log(s_ref[...]).T


def _sc_body(et_hbm, idx_hbm, lw_hbm, out_hbm,
             idx_v, lw_v, rows_v, out_v, gsem0, gsem1, osem0, osem1):
    c = lax.axis_index("c")
    s = lax.axis_index("s")
    gbase = s * GPP + c * G0
    ng = jnp.where(c == 0, G0, G1)
    npairs = ng // 2
    nbase = gbase * GSZ
    pltpu.sync_copy(idx_hbm.at[pl.ds(gbase, GMAX)], idx_v)
    pltpu.sync_copy(lw_hbm.at[pl.ds(gbase, GMAX)], lw_v)
    gsems = (gsem0, gsem1)
    osems = (osem0, osem1)

    pltpu.async_copy(et_hbm.at[idx_v.at[0]], rows_v.at[0], gsems[0])

    def pair(t, carry):
        for b in range(2):
            g = t * 2 + b
            nb = 1 - b

            @pl.when(g + 1 < ng)
            def _prefetch(g=g, nb=nb):
                pltpu.async_copy(et_hbm.at[idx_v.at[g + 1]],
                                 rows_v.at[nb], gsems[nb])

            pltpu.make_async_copy(et_hbm.at[pl.ds(0, GSZ * FAN)],
                                  rows_v.at[b], gsems[b]).wait()

            @pl.when(t >= 1)
            def _reclaim(b=b):
                pltpu.make_async_copy(out_v.at[b],
                                      out_hbm.at[pl.ds(0, GSZ)],
                                      osems[b]).wait()

            for k in range(GSZ):
                lwv = lw_v[g, pl.ds(k * FAN, FAN)]
                lwb = [_bcast_lane(lwv, r) for r in range(FAN)]

                def chunk(cc, _, k=k, b=b, lwb=lwb):
                    alo = jnp.zeros((16,), jnp.float32)
                    ahi = jnp.zeros((16,), jnp.float32)
                    for r in range(FAN):
                        vi = rows_v[b, k * FAN + r, pl.ds(cc * 16, 16)]
                        lo = lax.bitcast_convert_type(
                            vi << 16, jnp.float32)
                        hi = lax.bitcast_convert_type(
                            vi & jnp.int32(-65536), jnp.float32)
                        alo = alo + jnp.exp(lo + lwb[r])
                        ahi = ahi + jnp.exp(hi + lwb[r])
                    out_v[b, k, pl.ds(cc * 32, 16)] = alo
                    out_v[b, k, pl.ds(cc * 32 + 16, 16)] = ahi
                    return 0

                lax.fori_loop(0, B2 // 16, chunk, 0)

            pltpu.async_copy(out_v.at[b],
                             out_hbm.at[pl.ds(nbase + g * GSZ, GSZ)],
                             osems[b])
        return carry

    lax.fori_loop(0, npairs, pair, 0)

    pltpu.make_async_copy(out_v.at[0], out_hbm.at[pl.ds(0, GSZ)], osems[0]).wait()
    pltpu.make_async_copy(out_v.at[1], out_hbm.at[pl.ds(0, GSZ)], osems[1]).wait()


def _sc_call(et, idx3, lw3):
    mesh = plsc.VectorSubcoreMesh(core_axis_name="c", subcore_axis_name="s")
    f = pl.kernel(
        _sc_body,
        out_type=jax.ShapeDtypeStruct((N_PAD, B), jnp.float32),
        mesh=mesh,
        scratch_types=[
            pltpu.VMEM((GMAX, GSZ * FAN), jnp.int32),
            pltpu.VMEM((GMAX, GSZ * FAN), jnp.float32),
            pltpu.VMEM((2, GSZ * FAN, B2), jnp.int32),
            pltpu.VMEM((2, GSZ, B), jnp.float32),
            pltpu.SemaphoreType.DMA,
            pltpu.SemaphoreType.DMA,
            pltpu.SemaphoreType.DMA,
            pltpu.SemaphoreType.DMA,
        ],
    )
    return f(et, idx3, lw3)


_SIGMA = np.arange(B).reshape(B // 32, 2, 16).transpose(0, 2, 1).reshape(B)


def kernel(x, idxs, log_weights):
    xt16 = x.T.astype(jnp.bfloat16)
    eti = lax.bitcast_convert_type(
        xt16.reshape(NCH, B2, 2), jnp.int32)

    idx_p = jnp.pad(idxs, ((0, N_PAD - NN), (0, 0)))
    lw_p = jnp.pad(log_weights, ((0, N_PAD - NN), (0, 0)))
    idx3 = jnp.pad(idx_p.reshape(N_PAD * FAN // 128, 128),
                   ((0, GROW_PAD), (0, 0)))
    lw3 = jnp.pad(lw_p.reshape(N_PAD * FAN // 128, 128),
                  ((0, GROW_PAD), (0, 0)))

    s_pad = _sc_call(eti, idx3, lw3)

    lls_pad = pl.pallas_call(
        _log_t_body,
        grid=(N_PAD // R_C,),
        in_specs=[pl.BlockSpec((R_C, B), lambda i: (i, 0))],
        out_specs=pl.BlockSpec((B, R_C), lambda i: (0, i)),
        out_shape=jax.ShapeDtypeStruct((B, N_PAD), jnp.float32),
    )(s_pad)
    return lls_pad[_SIGMA, :NN]


# speedup vs baseline: 4.1126x; 4.1126x over previous
import jax
import jax.numpy as jnp
from jax import lax
from jax.experimental import pallas as pl
from jax.experimental.pallas import tpu as pltpu
from jax.experimental.pallas import tpu_sc as plsc

B = 256
NCH = 50000
NN = 10000
FAN = 16

NC = 2
NS = 16
NW = NC * NS
BPT = B // NW
NPAIRS = BPT // 2
BLK = 16
NBLK = NN // BLK
CBLK = 25
NCHUNK = NBLK // CBLK
CN = CBLK * BLK


def _log_body(s_ref, o_ref):
    o_ref[...] = jnp.log(s_ref[...])


def _sc_body(x_hbm, tt_hbm, out_hbm,
             xr0, xr1, tc0, tc1, ov0, ov1,
             xsem, tsem0, tsem1, osem0, osem1):
    c = lax.axis_index("c")
    s = lax.axis_index("s")
    wid = s * NC + c
    b0 = wid * BPT
    tcs = (tc0, tc1)
    ovs = (ov0, ov1)
    tsems = (tsem0, tsem1)
    osems = (osem0, osem1)

    def out_wait(cb):
        pltpu.make_async_copy(ovs[cb].at[0], out_hbm.at[0], osems[cb]).wait()
        pltpu.make_async_copy(ovs[cb].at[1], out_hbm.at[0], osems[cb]).wait()

    def do_chunk(cc, cb, obase):
        pltpu.make_async_copy(tt_hbm.at[pl.ds(0, CN)], tcs[cb],
                              tsems[cb]).wait()
        tcv = tcs[cb]
        ovv = ovs[cb]

        def blk_body(bl, _):
            acc0 = jnp.zeros((16,), jnp.float32)
            acc1 = jnp.zeros((16,), jnp.float32)
            for r in range(FAN):
                w = tcv[bl * FAN + r]
                idxv = w & jnp.int32(0xFFFF)
                lwv = lax.bitcast_convert_type(
                    w & jnp.int32(-65536), jnp.float32)
                rhi = idxv >> 4
                rlo = idxv & jnp.int32(15)
                g0 = plsc.load_gather(xr0, [rhi, rlo])
                g1 = plsc.load_gather(xr1, [rhi, rlo])
                acc0 = acc0 + jnp.exp(g0 + lwv)
                acc1 = acc1 + jnp.exp(g1 + lwv)
            ovv[0, pl.ds(bl * BLK, BLK)] = acc0
            ovv[1, pl.ds(bl * BLK, BLK)] = acc1
            return 0

        lax.fori_loop(0, CBLK, blk_body, 0)
        row = obase + cc
        pltpu.async_copy(ovv.at[0], out_hbm.at[row], osems[cb])
        pltpu.async_copy(ovv.at[1], out_hbm.at[row + NCHUNK], osems[cb])

    def pair_body(p, carry):
        bi = b0 + 2 * p
        obase = bi * NCHUNK
        pltpu.async_copy(tt_hbm.at[pl.ds(0, CN)], tcs[0], tsems[0])
        cx0 = pltpu.async_copy(x_hbm.at[bi], xr0, xsem)
        cx1 = pltpu.async_copy(x_hbm.at[bi + 1], xr1, xsem)
        cx0.wait()
        cx1.wait()

        def chunk_pair(t, carry2):
            for cb in range(2):
                cc = t * 2 + cb
                pltpu.async_copy(tt_hbm.at[pl.ds((cc + 1) * CN, CN)],
                                 tcs[1 - cb], tsems[1 - cb])

                @pl.when(t >= 1)
                def _reclaim(cb=cb):
                    out_wait(cb)

                do_chunk(cc, cb, obase)
            return carry2

        lax.fori_loop(0, NCHUNK // 2, chunk_pair, 0)

        out_wait(0)
        do_chunk(NCHUNK - 1, 0, obase)

        out_wait(0)
        out_wait(1)
        return carry

    lax.fori_loop(0, NPAIRS, pair_body, 0)


def _sc_call(x, tt):
    mesh = plsc.VectorSubcoreMesh(core_axis_name="c", subcore_axis_name="s")
    f = pl.kernel(
        _sc_body,
        out_type=jax.ShapeDtypeStruct((B * NN // CN, CN), jnp.float32),
        mesh=mesh,
        compiler_params=pltpu.CompilerParams(
            needs_layout_passes=False, use_tc_tiling_on_sc=False),
        scratch_types=[
            pltpu.VMEM((NCH // 16, 16), jnp.float32),
            pltpu.VMEM((NCH // 16, 16), jnp.float32),
            pltpu.VMEM((CN, FAN), jnp.int32),
            pltpu.VMEM((CN, FAN), jnp.int32),
            pltpu.VMEM((2, CN), jnp.float32),
            pltpu.VMEM((2, CN), jnp.float32),
            pltpu.SemaphoreType.DMA,
            pltpu.SemaphoreType.DMA,
            pltpu.SemaphoreType.DMA,
            pltpu.SemaphoreType.DMA,
            pltpu.SemaphoreType.DMA,
        ],
    )
    return f(x, tt)


def kernel(x, idxs, log_weights):
    lw16 = lax.bitcast_convert_type(
        log_weights.astype(jnp.bfloat16), jnp.uint16)
    packed = lax.bitcast_convert_type(
        (lw16.astype(jnp.uint32) << 16) | idxs.astype(jnp.uint32),
        jnp.int32)
    tt = packed.reshape(NBLK, BLK, FAN).swapaxes(1, 2).reshape(NN, FAN)

    s_flat = _sc_call(x.reshape(B, NCH // 16, 16), tt)

    lls_flat = pl.pallas_call(
        _log_body,
        grid=(8,),
        in_specs=[pl.BlockSpec((NBLK, 512), lambda i: (0, i))],
        out_specs=pl.BlockSpec((NBLK, 512), lambda i: (0, i)),
        out_shape=jax.ShapeDtypeStruct((NBLK, 4096), jnp.float32),
    )(s_flat.reshape(NBLK, 4096))
    return lls_flat.reshape(B, NN)


# speedup vs baseline: 5.7061x; 1.3875x over previous
import jax
import jax.numpy as jnp
from jax import lax
from jax.experimental import pallas as pl
from jax.experimental.pallas import tpu as pltpu
from jax.experimental.pallas import tpu_sc as plsc

B = 256
NCH = 50000
NN = 10000
FAN = 16

NC = 2
NS = 16
NW = NC * NS
BPT = B // NW
NPAIRS = BPT // 2
BLK = 16
NBLK = NN // BLK
CBLK = 25
NCHUNK = NBLK // CBLK
CN = CBLK * BLK


def _log_body(s_ref, o_ref):
    o_ref[...] = jnp.log(s_ref[...])


def _sc_body(x_hbm, tt_hbm, out_hbm,
             xr0, xr1, tc0, tc1, ova0, ovb0, ova1, ovb1,
             xsem, tsem0, tsem1, osem0, osem1):
    c = lax.axis_index("c")
    s = lax.axis_index("s")
    wid = s * NC + c
    b0 = wid * BPT
    tcs = (tc0, tc1)
    ovs = ((ova0, ovb0), (ova1, ovb1))
    tsems = (tsem0, tsem1)
    osems = (osem0, osem1)
    lane = lax.iota(jnp.int32, 16)
    cols = [jnp.full((16,), r, jnp.int32) for r in range(FAN)]

    def out_wait(cb):
        pltpu.make_async_copy(ovs[cb][0], out_hbm.at[pl.ds(0, CN)],
                              osems[cb]).wait()
        pltpu.make_async_copy(ovs[cb][1], out_hbm.at[pl.ds(0, CN)],
                              osems[cb]).wait()

    def do_chunk(cc, cb, obase):
        pltpu.make_async_copy(tt_hbm.at[pl.ds(0, CN)], tcs[cb],
                              tsems[cb]).wait()
        tcv = tcs[cb]
        ova, ovb = ovs[cb]

        def blk_body(bl, _):
            rowv = bl * BLK + lane
            acc0 = jnp.zeros((16,), jnp.float32)
            acc1 = jnp.zeros((16,), jnp.float32)
            for r in range(FAN):
                w = plsc.load_gather(tcv, [rowv, cols[r]])
                idxv = w & jnp.int32(0xFFFF)
                lwv = lax.bitcast_convert_type(
                    w & jnp.int32(-65536), jnp.float32)
                g0 = plsc.load_gather(xr0, [idxv])
                g1 = plsc.load_gather(xr1, [idxv])
                acc0 = acc0 + jnp.exp(g0 + lwv)
                acc1 = acc1 + jnp.exp(g1 + lwv)
            ova[pl.ds(bl * BLK, BLK)] = acc0
            ovb[pl.ds(bl * BLK, BLK)] = acc1
            return 0

        lax.fori_loop(0, CBLK, blk_body, 0)
        base = obase + cc * CN
        pltpu.async_copy(ova, out_hbm.at[pl.ds(base, CN)], osems[cb])
        pltpu.async_copy(ovb, out_hbm.at[pl.ds(base + NN, CN)], osems[cb])

    def pair_body(p, carry):
        bi = b0 + 2 * p
        obase = bi * NN
        pltpu.async_copy(tt_hbm.at[pl.ds(0, CN)], tcs[0], tsems[0])
        cx0 = pltpu.async_copy(x_hbm.at[bi], xr0, xsem)
        cx1 = pltpu.async_copy(x_hbm.at[bi + 1], xr1, xsem)
        cx0.wait()
        cx1.wait()

        def chunk_pair(t, carry2):
            for cb in range(2):
                cc = t * 2 + cb
                pltpu.async_copy(tt_hbm.at[pl.ds((cc + 1) * CN, CN)],
                                 tcs[1 - cb], tsems[1 - cb])

                @pl.when(t >= 1)
                def _reclaim(cb=cb):
                    out_wait(cb)

                do_chunk(cc, cb, obase)
            return carry2

        lax.fori_loop(0, NCHUNK // 2, chunk_pair, 0)

        out_wait(0)
        do_chunk(NCHUNK - 1, 0, obase)

        out_wait(0)
        out_wait(1)
        return carry

    lax.fori_loop(0, NPAIRS, pair_body, 0)


def _sc_call(x, tt):
    mesh = plsc.VectorSubcoreMesh(core_axis_name="c", subcore_axis_name="s")
    f = pl.kernel(
        _sc_body,
        out_type=jax.ShapeDtypeStruct((B * NN,), jnp.float32),
        mesh=mesh,
        compiler_params=pltpu.CompilerParams(
            needs_layout_passes=False, use_tc_tiling_on_sc=False),
        scratch_types=[
            pltpu.VMEM((NCH,), jnp.float32),
            pltpu.VMEM((NCH,), jnp.float32),
            pltpu.VMEM((CN, FAN), jnp.int32),
            pltpu.VMEM((CN, FAN), jnp.int32),
            pltpu.VMEM((CN,), jnp.float32),
            pltpu.VMEM((CN,), jnp.float32),
            pltpu.VMEM((CN,), jnp.float32),
            pltpu.VMEM((CN,), jnp.float32),
            pltpu.SemaphoreType.DMA,
            pltpu.SemaphoreType.DMA,
            pltpu.SemaphoreType.DMA,
            pltpu.SemaphoreType.DMA,
            pltpu.SemaphoreType.DMA,
        ],
    )
    return f(x, tt)


def kernel(x, idxs, log_weights):
    lw16 = lax.bitcast_convert_type(
        log_weights.astype(jnp.bfloat16), jnp.uint16)
    tt = lax.bitcast_convert_type(
        (lw16.astype(jnp.uint32) << 16) | idxs.astype(jnp.uint32),
        jnp.int32)

    s_flat = _sc_call(x, tt)

    lls_flat = pl.pallas_call(
        _log_body,
        grid=(8,),
        in_specs=[pl.BlockSpec((NBLK, 512), lambda i: (0, i))],
        out_specs=pl.BlockSpec((NBLK, 512), lambda i: (0, i)),
        out_shape=jax.ShapeDtypeStruct((NBLK, 4096), jnp.float32),
    )(s_flat.reshape(NBLK, 4096))
    return lls_flat.reshape(B, NN)


# speedup vs baseline: 5.7315x; 1.0045x over previous
import jax
import jax.numpy as jnp
from jax import lax
from jax.experimental import pallas as pl
from jax.experimental.pallas import tpu as pltpu
from jax.experimental.pallas import tpu_sc as plsc

B = 256
NCH = 50000
NN = 10000
FAN = 16

NC = 2
NS = 16
NW = NC * NS
BPT = B // NW
NPAIRS = BPT // 2
BLK = 16
NBLK = NN // BLK
CBLK = 25
NCHUNK = NBLK // CBLK
CN = CBLK * BLK


def _log_body(s_ref, o_ref):
    o_ref[...] = jnp.log(s_ref[...])


def _sc_body(x_hbm, tt_hbm, out_hbm,
             xr0, xr1, tc0, tc1, ova0, ovb0, ova1, ovb1,
             xsem, tsem0, tsem1, osem0, osem1):
    c = lax.axis_index("c")
    s = lax.axis_index("s")
    wid = s * NC + c
    b0 = wid * BPT
    tcs = (tc0, tc1)
    ovs = ((ova0, ovb0), (ova1, ovb1))
    tsems = (tsem0, tsem1)
    osems = (osem0, osem1)
    lane = lax.iota(jnp.int32, 16)
    cols = [jnp.full((16,), r, jnp.int32) for r in range(FAN)]

    def out_wait(cb):
        pltpu.make_async_copy(ovs[cb][0], out_hbm.at[0], osems[cb]).wait()
        pltpu.make_async_copy(ovs[cb][1], out_hbm.at[0], osems[cb]).wait()

    def do_chunk(cc, cb, obase):
        pltpu.make_async_copy(tt_hbm.at[pl.ds(0, CN)], tcs[cb],
                              tsems[cb]).wait()
        tcv = tcs[cb]
        ova, ovb = ovs[cb]

        def blk_body(bl, _):
            rowv = bl * BLK + lane
            acc0 = jnp.zeros((16,), jnp.float32)
            acc1 = jnp.zeros((16,), jnp.float32)
            for r in range(FAN):
                w = plsc.load_gather(tcv, [rowv, cols[r]])
                idxv = w & jnp.int32(0xFFFF)
                lwv = lax.bitcast_convert_type(
                    w & jnp.int32(-65536), jnp.float32)
                g0 = plsc.load_gather(xr0, [idxv])
                g1 = plsc.load_gather(xr1, [idxv])
                acc0 = acc0 + jnp.exp(g0 + lwv)
                acc1 = acc1 + jnp.exp(g1 + lwv)
            ova[pl.ds(bl * BLK, BLK)] = acc0
            ovb[pl.ds(bl * BLK, BLK)] = acc1
            return 0

        lax.fori_loop(0, CBLK, blk_body, 0)
        row = obase + cc
        pltpu.async_copy(ova, out_hbm.at[row], osems[cb])
        pltpu.async_copy(ovb, out_hbm.at[row + NCHUNK], osems[cb])

    def pair_body(p, carry):
        bi = b0 + 2 * p
        obase = bi * NCHUNK
        pltpu.async_copy(tt_hbm.at[pl.ds(0, CN)], tcs[0], tsems[0])
        cx0 = pltpu.async_copy(x_hbm.at[bi], xr0, xsem)
        cx1 = pltpu.async_copy(x_hbm.at[bi + 1], xr1, xsem)
        cx0.wait()
        cx1.wait()

        def chunk_pair(t, carry2):
            for cb in range(2):
                cc = t * 2 + cb
                pltpu.async_copy(tt_hbm.at[pl.ds((cc + 1) * CN, CN)],
                                 tcs[1 - cb], tsems[1 - cb])

                @pl.when(t >= 1)
                def _reclaim(cb=cb):
                    out_wait(cb)

                do_chunk(cc, cb, obase)
            return carry2

        lax.fori_loop(0, NCHUNK // 2, chunk_pair, 0)

        out_wait(0)
        do_chunk(NCHUNK - 1, 0, obase)

        out_wait(0)
        out_wait(1)
        return carry

    lax.fori_loop(0, NPAIRS, pair_body, 0)


def _sc_call(x, tt):
    mesh = plsc.VectorSubcoreMesh(core_axis_name="c", subcore_axis_name="s")
    f = pl.kernel(
        _sc_body,
        out_type=jax.ShapeDtypeStruct((B * NN // CN, CN), jnp.float32),
        mesh=mesh,
        compiler_params=pltpu.CompilerParams(
            needs_layout_passes=False, use_tc_tiling_on_sc=False),
        scratch_types=[
            pltpu.VMEM((NCH,), jnp.float32),
            pltpu.VMEM((NCH,), jnp.float32),
            pltpu.VMEM((CN, FAN), jnp.int32),
            pltpu.VMEM((CN, FAN), jnp.int32),
            pltpu.VMEM((CN,), jnp.float32),
            pltpu.VMEM((CN,), jnp.float32),
            pltpu.VMEM((CN,), jnp.float32),
            pltpu.VMEM((CN,), jnp.float32),
            pltpu.SemaphoreType.DMA,
            pltpu.SemaphoreType.DMA,
            pltpu.SemaphoreType.DMA,
            pltpu.SemaphoreType.DMA,
            pltpu.SemaphoreType.DMA,
        ],
    )
    return f(x, tt)


def kernel(x, idxs, log_weights):
    lw16 = lax.bitcast_convert_type(
        log_weights.astype(jnp.bfloat16), jnp.uint16)
    tt = lax.bitcast_convert_type(
        (lw16.astype(jnp.uint32) << 16) | idxs.astype(jnp.uint32),
        jnp.int32)

    s2 = _sc_call(x, tt)

    lls2 = pl.pallas_call(
        _log_body,
        grid=(8,),
        in_specs=[pl.BlockSpec((B * NN // CN // 8, CN), lambda i: (i, 0))],
        out_specs=pl.BlockSpec((B * NN // CN // 8, CN), lambda i: (i, 0)),
        out_shape=jax.ShapeDtypeStruct((B * NN // CN, CN), jnp.float32),
    )(s2)
    return lls2.reshape(B, NN)


# speedup vs baseline: 6.8602x; 1.1969x over previous
import jax
import jax.numpy as jnp
from jax import lax
from jax.experimental import pallas as pl
from jax.experimental.pallas import tpu as pltpu
from jax.experimental.pallas import tpu_sc as plsc

B = 256
NCH = 50000
NN = 10000
FAN = 16

NC = 2
NS = 16
NW = NC * NS
NPW = 320
N_PAD = NW * NPW
GSZ = 8
GROUPS = NPW // GSZ

R_C = 1280

_BCAST_DNUMS = lax.GatherDimensionNumbers(
    offset_dims=(), collapsed_slice_dims=(0,), start_index_map=(0,))


def _bcast_lane(v, r):
    idx = jnp.full((16, 1), r, jnp.int32)
    return lax.gather(v, idx, _BCAST_DNUMS, (1,),
                      mode=lax.GatherScatterMode.PROMISE_IN_BOUNDS)


def _log_t_body(s_ref, o_ref):
    o_ref[...] = jnp.log(s_ref[...]).T


def _sc_body(et_hbm, idx_hbm, lw_hbm, out_hbm,
             idx_v, lw_v, rows_v, out_v, gsem0, gsem1, osem0, osem1):
    wid = lax.axis_index("s") * NC + lax.axis_index("c")
    pltpu.sync_copy(idx_hbm.at[wid], idx_v)
    pltpu.sync_copy(lw_hbm.at[wid], lw_v)
    base = wid * NPW
    gsems = (gsem0, gsem1)
    osems = (osem0, osem1)

    pltpu.async_copy(et_hbm.at[idx_v.at[0]], rows_v.at[0], gsems[0])

    def pair(t, carry):
        for b in range(2):
            g = t * 2 + b
            nb = 1 - b

            @pl.when(g + 1 < GROUPS)
            def _prefetch(g=g, nb=nb):
                pltpu.async_copy(et_hbm.at[idx_v.at[g + 1]],
                                 rows_v.at[nb], gsems[nb])

            pltpu.make_async_copy(et_hbm.at[pl.ds(0, GSZ * FAN)],
                                  rows_v.at[b], gsems[b]).wait()

            @pl.when(t >= 1)
            def _reclaim(b=b):
                pltpu.make_async_copy(out_v.at[b],
                                      out_hbm.at[pl.ds(0, GSZ)],
                                      osems[b]).wait()

            for k in range(GSZ):
                lwv = lw_v[g, pl.ds(k * FAN, FAN)]
                lwb = [_bcast_lane(lwv, r) for r in range(FAN)]

                def chunk(c, _, k=k, b=b, lwb=lwb):
                    acc = jnp.zeros((16,), jnp.float32)
                    for r in range(FAN):
                        acc = acc + jnp.exp(
                            lwb[r] + rows_v[b, k * FAN + r,
                                            pl.ds(c * 16, 16)])
                    out_v[b, k, pl.ds(c * 16, 16)] = acc
                    return 0

                lax.fori_loop(0, B // 16, chunk, 0)

            pltpu.async_copy(out_v.at[b],
                             out_hbm.at[pl.ds(base + g * GSZ, GSZ)],
                             osems[b])
        return carry

    lax.fori_loop(0, GROUPS // 2, pair, 0)

    pltpu.make_async_copy(out_v.at[0], out_hbm.at[pl.ds(0, GSZ)], osems[0]).wait()
    pltpu.make_async_copy(out_v.at[1], out_hbm.at[pl.ds(0, GSZ)], osems[1]).wait()


def _sc_call(et, idx3, lw3):
    mesh = plsc.VectorSubcoreMesh(core_axis_name="c", subcore_axis_name="s")
    f = pl.kernel(
        _sc_body,
        out_type=jax.ShapeDtypeStruct((N_PAD, B), jnp.float32),
        mesh=mesh,
        scratch_types=[
            pltpu.VMEM((GROUPS, GSZ * FAN), jnp.int32),
            pltpu.VMEM((GROUPS, GSZ * FAN), jnp.float32),
            pltpu.VMEM((2, GSZ * FAN, B), jnp.float32),
            pltpu.VMEM((2, GSZ, B), jnp.float32),
            pltpu.SemaphoreType.DMA,
            pltpu.SemaphoreType.DMA,
            pltpu.SemaphoreType.DMA,
            pltpu.SemaphoreType.DMA,
        ],
    )
    return f(et, idx3, lw3)


def kernel(x, idxs, log_weights):
    xt = x.T

    idx_p = jnp.pad(idxs, ((0, N_PAD - NN), (0, 0)))
    lw_p = jnp.pad(log_weights, ((0, N_PAD - NN), (0, 0)))
    idx3 = idx_p.reshape(NW, GROUPS, GSZ * FAN)
    lw3 = lw_p.reshape(NW, GROUPS, GSZ * FAN)

    s_pad = _sc_call(xt, idx3, lw3)

    lls_pad = pl.pallas_call(
        _log_t_body,
        grid=(N_PAD // R_C,),
        in_specs=[pl.BlockSpec((R_C, B), lambda i: (i, 0))],
        out_specs=pl.BlockSpec((B, R_C), lambda i: (0, i)),
        out_shape=jax.ShapeDtypeStruct((B, N_PAD), jnp.float32),
    )(s_pad)
    return lls_pad[:, :NN]
